# Initial kernel scaffold; baseline (speedup 1.0000x reference)
import jax
import jax.numpy as jnp
from jax.experimental import pallas as pl


def kernel(x, edge_index, batch, params):
    raise NotImplementedError("write your pallas kernel here")


# speedup vs baseline: 1.0000x; 1.0000x over previous
import jax
import jax.numpy as jnp
from jax.experimental import pallas as pl

N = 50000
HIDDEN = 128
HEADS = 4
NUM_GRAPHS = 256


def _fc_head_kernel(g_ref, w1_ref, b1_ref, w2_ref, b2_ref, o_ref):
    g = g_ref[...]
    h = jnp.maximum(
        jnp.dot(g, w1_ref[...], preferred_element_type=jnp.float32) + b1_ref[...], 0.0
    )
    o_ref[...] = jnp.dot(h, w2_ref[...], preferred_element_type=jnp.float32) + b2_ref[...]


def _gat_layer(x, edge_index, prm, heads, out_ch, concat):
    n = x.shape[0]
    loop = jnp.arange(n, dtype=edge_index.dtype)
    src = jnp.concatenate([edge_index[0], loop])
    dst = jnp.concatenate([edge_index[1], loop])
    h = (x @ prm["W"]).reshape(n, heads, out_ch)
    a_src = jnp.sum(h * prm["att_src"], axis=-1)
    a_dst = jnp.sum(h * prm["att_dst"], axis=-1)
    alpha = jax.nn.leaky_relu(a_src[src] + a_dst[dst], negative_slope=0.2)
    amax = jax.ops.segment_max(alpha, dst, num_segments=n)
    ex = jnp.exp(alpha - amax[dst])
    den = jax.ops.segment_sum(ex, dst, num_segments=n)
    att = ex / (den[dst] + 1e-16)
    out = jax.ops.segment_sum(h[src] * att[:, :, None], dst, num_segments=n)
    if concat:
        out = out.reshape(n, heads * out_ch)
    else:
        out = jnp.mean(out, axis=1)
    return out + prm["bias"]


def kernel(x, edge_index, batch, params):
    h = jax.nn.relu(x @ params["W_embed"] + params["b_embed"])
    cfgs = [(HEADS, HIDDEN, True), (HEADS, HIDDEN, True), (1, HIDDEN, False)]
    for i, (heads, oc, concat) in enumerate(cfgs):
        res = h
        h = _gat_layer(h, edge_index, params["gat"][i], heads, oc, concat)
        bn = params["bn"][i]
        h = h * (bn["gamma"] / jnp.sqrt(1.0 + 1e-5)) + bn["beta"]
        h = jax.nn.relu(h)
        if i > 0 and h.shape[-1] == res.shape[-1]:
            h = h + res
    ones = jnp.ones((h.shape[0],), jnp.float32)
    cnt = jax.ops.segment_sum(ones, batch, num_segments=NUM_GRAPHS)
    mean_p = jax.ops.segment_sum(h, batch, num_segments=NUM_GRAPHS) / jnp.maximum(cnt, 1.0)[:, None]
    max_p = jax.ops.segment_max(h, batch, num_segments=NUM_GRAPHS)
    g = jnp.concatenate([mean_p, max_p], axis=-1)
    fc = params["fc"]
    return pl.pallas_call(
        _fc_head_kernel,
        out_shape=jax.ShapeDtypeStruct((NUM_GRAPHS, 128), jnp.float32),
    )(g, fc["W1"], fc["b1"], fc["W2"], fc["b2"])


# speedup vs baseline: 17.5722x; 17.5720x over previous
import dataclasses
import functools

import jax
import jax.numpy as jnp
from jax import lax
from jax.experimental import pallas as pl
from jax.experimental.pallas import tpu as pltpu
from jax.experimental.pallas import tpu_sc as plsc

N = 50000
E = 800000
IN_FEATURES = 78
HIDDEN = 128
HEADS = 4
OUT_DIM = 128
NUM_GRAPHS = 256

NP = 51200
ET = E + N
EP = 851968
NWT = NP // 32
GP = 272


@functools.cache
def _mesh():
    return plsc.VectorSubcoreMesh(core_axis_name="c", subcore_axis_name="s")


@functools.cache
def _sc_params():
    cp = pltpu.CompilerParams()
    if "needs_layout_passes" in pltpu.CompilerParams.__dataclass_fields__:
        cp = dataclasses.replace(cp, needs_layout_passes=False)
    return cp


def _bcast(v, j):
    idx = jnp.full((16,), 0, jnp.int32) + j
    return v.at[idx].get(mode="promise_in_bounds")


BB = 32


def _msg_body(D, H, hwe, adst, srcs, dsts, offs, zf, agg,
              offv, sidx, didx, lbuf, gbuf, zfb, adstb, dsem):
    c = lax.axis_index("c")
    s = lax.axis_index("s")
    w = s * 2 + c
    CH = D // 16
    DEXT = D + 128
    iota = lax.iota(jnp.int32, 16)

    pltpu.sync_copy(offs, offv)
    pltpu.sync_copy(zf, zfb)
    pltpu.sync_copy(adst.at[pl.ds(w * (NWT // 8), NWT // 8)], adstb)

    @pl.loop(0, NWT // 8)
    def _(z):
        pltpu.sync_copy(zfb, agg.at[pl.ds(w * NWT + z * 8, 8)])

    widx = jnp.full((16,), 0, jnp.int32) + w
    lo = plsc.load_gather(offv, [widx])[0]
    hi = plsc.load_gather(offv, [widx + 1])[0]
    al = (lo // BB) * BB
    nb = (jnp.maximum(hi - al, 0) + BB - 1) // BB
    base_row = w * NWT

    @pl.loop(0, nb)
    def _(j):
        st = al + j * BB
        pltpu.sync_copy(srcs.at[pl.ds(st, BB)], sidx)
        pltpu.sync_copy(dsts.at[pl.ds(st, BB)], didx)
        pltpu.sync_copy(hwe.at[sidx], gbuf)

        for q in range(BB // 16):
            dv = didx[pl.ds(q * 16, 16)]
            lv = jnp.clip(dv - base_row, 0, NWT - 1)
            lbuf[pl.ds(q * 16, 16)] = lv + base_row
            rv = lv >> 3
            cv = (lv & 7) * 16
            for t in range(16):
                e = q * 16 + t
                ge = st + e
                valid = jnp.logical_and(ge >= lo, ge < hi)
                vf = jnp.where(valid, jnp.float32(1.0), jnp.float32(0.0))
                bdv = plsc.load_gather(adstb, [_bcast(rv, t), _bcast(cv, t) + iota])
                alpha = gbuf[e, pl.ds(D, 16)] + bdv
                alpha = jnp.where(alpha > 0, alpha, alpha * jnp.float32(0.2))
                exv = jnp.exp(alpha) * vf
                gbuf[e, pl.ds(D, 16)] = exv
                for h in range(H):
                    wh = _bcast(exv, h)
                    for k in range(CH // H):
                        col = (h * (CH // H) + k) * 16
                        gbuf[e, pl.ds(col, 16)] = gbuf[e, pl.ds(col, 16)] * wh

        pltpu.async_copy(gbuf, agg.at[lbuf], dsem, add=True).wait()


def _msg_pass(D, H, hwe, adst, srcs, dsts, offs):
    adst = adst.reshape(NP // 8, 8 * 16)[:, :128]
    DEXT = D + 128
    zf = jnp.zeros((8, DEXT), jnp.float32)
    kern = pl.kernel(
        functools.partial(_msg_body, D, H),
        out_type=jax.ShapeDtypeStruct((NP, DEXT), jnp.float32),
        mesh=_mesh(),
        compiler_params=_sc_params(),
        scratch_types=[
            pltpu.VMEM((48,), jnp.int32),
            pltpu.VMEM((BB,), jnp.int32),
            pltpu.VMEM((BB,), jnp.int32),
            pltpu.VMEM((BB,), jnp.int32),
            pltpu.VMEM((BB, DEXT), jnp.float32),
            pltpu.VMEM((8, DEXT), jnp.float32),
            pltpu.VMEM((NWT // 8, 128), jnp.float32),
            pltpu.SemaphoreType.DMA,
        ],
    )
    return kern(hwe, adst, srcs, dsts, offs, zf)


NW = NP // 32
BC = 64


def _pool_body(agg3, batch, bst, zsum, zmax, zcnt,
               sum_p, max_p, cnt_p,
               rows, bidx, bstb, sumt, maxt, cntt):
    c = lax.axis_index("c")
    s = lax.axis_index("s")
    w = s * 2 + c
    iota = lax.iota(jnp.int32, 16)

    pltpu.sync_copy(zsum, sumt)
    pltpu.sync_copy(zmax, maxt)
    pltpu.sync_copy(zcnt, cntt)
    pltpu.sync_copy(bst, bstb)

    base = w * NW
    csixteenth = jnp.full((16,), 0.0625, jnp.float32)

    @pl.loop(0, NW // BC)
    def _(j):
        st = base + j * BC
        pltpu.sync_copy(agg3.at[pl.ds(st, BC)], rows)
        pltpu.sync_copy(batch.at[pl.ds(st, BC)], bidx)

        for q in range(BC // 16):
            bv = bidx[pl.ds(q * 16, 16)]
            for t in range(16):
                e = q * 16 + t
                d = _bcast(rows[e, pl.ds(128, 16)], 0) + jnp.float32(1e-16)
                g16 = _bcast(bv, t)
                for k in range(8):
                    cs = pl.ds(k * 16, 16)
                    y = rows[e, cs] / d
                    y = (y + bstb[0, cs]) * bstb[1, cs]
                    y = jnp.maximum(y + bstb[2, cs], jnp.float32(0.0))
                    colv = iota + k * 16
                    plsc.addupdate_scatter(sumt, [g16, colv], y)
                    cur = plsc.load_gather(maxt, [g16, colv])
                    plsc.store_scatter(maxt, [g16, colv], jnp.maximum(cur, y))
                plsc.addupdate_scatter(cntt, [g16, iota], csixteenth)

    pltpu.sync_copy(sumt, sum_p.at[w])
    pltpu.sync_copy(maxt, max_p.at[w])
    pltpu.sync_copy(cntt, cnt_p.at[w])


def _pool_pass(agg3, batch, bst, zsum, zmax, zcnt):
    kern = pl.kernel(
        _pool_body,
        out_type=[
            jax.ShapeDtypeStruct((32, GP, 128), jnp.float32),
            jax.ShapeDtypeStruct((32, GP, 128), jnp.float32),
            jax.ShapeDtypeStruct((32, GP, 16), jnp.float32),
        ],
        mesh=_mesh(),
        compiler_params=_sc_params(),
        scratch_types=[
            pltpu.VMEM((BC, 256), jnp.float32),
            pltpu.VMEM((BC,), jnp.int32),
            pltpu.VMEM((3, 128), jnp.float32),
            pltpu.VMEM((GP, 128), jnp.float32),
            pltpu.VMEM((GP, 128), jnp.float32),
            pltpu.VMEM((GP, 16), jnp.float32),
        ],
    )
    return kern(agg3, batch, bst, zsum, zmax, zcnt)


def _k1_body(x_ref, we_ref, be_ref, w1_ref, a1s_ref, a1d_ref,
             hwe_ref, ad_ref):
    h0 = jnp.maximum(
        jnp.dot(x_ref[...], we_ref[...], preferred_element_type=jnp.float32)
        + be_ref[...], 0.0)
    hw = jnp.dot(h0, w1_ref[...], preferred_element_type=jnp.float32)
    asrc = jnp.dot(hw, a1s_ref[...], preferred_element_type=jnp.float32)
    hwe_ref[...] = jnp.concatenate([hw, asrc], axis=1)
    ad_ref[...] = jnp.dot(hw, a1d_ref[...], preferred_element_type=jnp.float32)


def _k23_body(has_res, agg_ref, denb_ref, res_ref, bias_ref, g1_ref,
              g2_ref, w_ref, as_m_ref, ad_m_ref, hp_ref, hwe_ref, ad_ref):
    b = agg_ref.shape[0]
    nh = agg_ref.shape[1] // HIDDEN
    agg3 = agg_ref[...].reshape(b, nh, HIDDEN)
    den3 = denb_ref[...][:, :nh].reshape(b, nh, 1)
    out = (agg3 / (den3 + 1e-16)).reshape(b, nh * HIDDEN)
    hp = jnp.maximum((out + bias_ref[...]) * g1_ref[...] + g2_ref[...], 0.0)
    if has_res:
        hp = hp + res_ref[...]
    hp_ref[...] = hp
    hw = jnp.dot(hp, w_ref[...], preferred_element_type=jnp.float32)
    asrc = jnp.dot(hw, as_m_ref[...], preferred_element_type=jnp.float32)
    hwe_ref[...] = jnp.concatenate([hw, asrc], axis=1)
    ad_ref[...] = jnp.dot(hw, ad_m_ref[...], preferred_element_type=jnp.float32)


def _k4_body(sum_ref, max_ref, cnt_ref, w1_ref, b1_ref, w2_ref, b2_ref, o_ref):
    stot = jnp.sum(sum_ref[...], axis=0)[:NUM_GRAPHS]
    mtot = jnp.max(max_ref[...], axis=0)[:NUM_GRAPHS]
    cnt = jnp.sum(cnt_ref[...], axis=0)[:NUM_GRAPHS]
    cnt = jnp.sum(cnt, axis=1, keepdims=True)
    mean = stot / jnp.maximum(cnt, 1.0)
    g = jnp.concatenate([mean, mtot], axis=1)
    h = jnp.maximum(
        jnp.dot(g, w1_ref[...], preferred_element_type=jnp.float32) + b1_ref[...],
        0.0)
    o_ref[...] = jnp.dot(h, w2_ref[...], preferred_element_type=jnp.float32) + b2_ref[...]


_BR = 256


def _tc_k1(x_pad, we, be, w1, a1s, a1d):
    return pl.pallas_call(
        _k1_body,
        grid=(NP // _BR,),
        in_specs=[
            pl.BlockSpec((_BR, 128), lambda i: (i, 0)),
            pl.BlockSpec((128, 128), lambda i: (0, 0)),
            pl.BlockSpec((1, 128), lambda i: (0, 0)),
            pl.BlockSpec((128, 4 * HIDDEN), lambda i: (0, 0)),
            pl.BlockSpec((4 * HIDDEN, 128), lambda i: (0, 0)),
            pl.BlockSpec((4 * HIDDEN, 16), lambda i: (0, 0)),
        ],
        out_specs=[
            pl.BlockSpec((_BR, 4 * HIDDEN + 128), lambda i: (i, 0)),
            pl.BlockSpec((_BR, 16), lambda i: (i, 0)),
        ],
        out_shape=[
            jax.ShapeDtypeStruct((NP, 4 * HIDDEN + 128), jnp.float32),
            jax.ShapeDtypeStruct((NP, 16), jnp.float32),
        ],
    )(x_pad, we, be, w1, a1s, a1d)


def _tc_k23(has_res, d_in, d_out, agge, res, bias, g1, g2, w, as_m, ad_m):
    di_blk = d_in // 128
    return pl.pallas_call(
        functools.partial(_k23_body, has_res),
        grid=(NP // _BR,),
        in_specs=[
            pl.BlockSpec((_BR, d_in), lambda i: (i, 0)),
            pl.BlockSpec((_BR, 128), lambda i, n=di_blk: (i, n)),
            pl.BlockSpec((_BR, d_in), lambda i: (i, 0)),
            pl.BlockSpec((1, d_in), lambda i: (0, 0)),
            pl.BlockSpec((1, d_in), lambda i: (0, 0)),
            pl.BlockSpec((1, d_in), lambda i: (0, 0)),
            pl.BlockSpec((d_in, d_out), lambda i: (0, 0)),
            pl.BlockSpec((d_out, 128), lambda i: (0, 0)),
            pl.BlockSpec((d_out, 16), lambda i: (0, 0)),
        ],
        out_specs=[
            pl.BlockSpec((_BR, d_in), lambda i: (i, 0)),
            pl.BlockSpec((_BR, d_out + 128), lambda i: (i, 0)),
            pl.BlockSpec((_BR, 16), lambda i: (i, 0)),
        ],
        out_shape=[
            jax.ShapeDtypeStruct((NP, d_in), jnp.float32),
            jax.ShapeDtypeStruct((NP, d_out + 128), jnp.float32),
            jax.ShapeDtypeStruct((NP, 16), jnp.float32),
        ],
    )(agge, agge, res, bias, g1, g2, w, as_m, ad_m)


def _tc_k4(sum_p, max_p, cnt_p, w1, b1, w2, b2):
    return pl.pallas_call(
        _k4_body,
        out_shape=jax.ShapeDtypeStruct((NUM_GRAPHS, OUT_DIM), jnp.float32),
    )(sum_p, max_p, cnt_p, w1, b1, w2, b2)


def _att_mat(att, width=16):
    h, c = att.shape
    eye = jnp.eye(h, width, dtype=jnp.float32)
    return jnp.einsum("hc,hj->hcj", att, eye).reshape(h * c, width)


def kernel(x, edge_index, batch, params):
    f32 = jnp.float32
    i32 = jnp.int32

    loop = jnp.arange(N, dtype=i32)
    pad = jnp.full((EP - ET,), N, dtype=i32)
    src = jnp.concatenate([edge_index[0].astype(i32), loop, pad])
    dst = jnp.concatenate([edge_index[1].astype(i32), loop, pad])
    order = jnp.argsort(dst)
    srcs = src[order]
    dsts = dst[order]
    offs = jnp.searchsorted(
        dsts, jnp.arange(48, dtype=i32) * NWT, side="left").astype(i32)

    x_pad = jnp.zeros((NP, 128), f32).at[:N, :IN_FEATURES].set(x)
    batch_pad = jnp.full((NP,), NUM_GRAPHS, i32).at[:N].set(batch.astype(i32))

    zsum = jnp.zeros((GP, 128), f32)
    zmax = jnp.full((GP, 128), -1e30, f32)
    zcnt = jnp.zeros((GP, 16), f32)

    p = params
    we = jnp.zeros((128, 128), f32).at[:IN_FEATURES].set(p["W_embed"])
    be = p["b_embed"].reshape(1, 128)

    gat = p["gat"]
    bn = p["bn"]
    inv = 1.0 / jnp.sqrt(1.0 + 1e-5)

    hwe1, ad1 = _tc_k1(x_pad, we, be, gat[0]["W"],
                       _att_mat(gat[0]["att_src"], 128),
                       _att_mat(gat[0]["att_dst"]))
    agge1 = _msg_pass(512, 4, hwe1, ad1, srcs, dsts, offs)

    h1p, hwe2, ad2 = _tc_k23(
        False, 512, 512, agge1, hwe1,
        gat[0]["bias"].reshape(1, 512), (bn[0]["gamma"] * inv).reshape(1, 512),
        bn[0]["beta"].reshape(1, 512), gat[1]["W"],
        _att_mat(gat[1]["att_src"], 128), _att_mat(gat[1]["att_dst"]))
    agge2 = _msg_pass(512, 4, hwe2, ad2, srcs, dsts, offs)

    _, hwe3, ad3 = _tc_k23(
        True, 512, 128, agge2, h1p,
        gat[1]["bias"].reshape(1, 512), (bn[1]["gamma"] * inv).reshape(1, 512),
        bn[1]["beta"].reshape(1, 512), gat[2]["W"],
        _att_mat(gat[2]["att_src"], 128), _att_mat(gat[2]["att_dst"]))
    agge3 = _msg_pass(128, 1, hwe3, ad3, srcs, dsts, offs)

    bst = jnp.stack([
        gat[2]["bias"], bn[2]["gamma"] * inv, bn[2]["beta"]]).astype(f32)
    sum_p, max_p, cnt_p = _pool_pass(agge3, batch_pad, bst, zsum, zmax, zcnt)

    fc = p["fc"]
    return _tc_k4(sum_p, max_p, cnt_p, fc["W1"], fc["b1"].reshape(1, OUT_DIM),
                  fc["W2"], fc["b2"].reshape(1, OUT_DIM))


# speedup vs baseline: 20.9236x; 1.1907x over previous
import dataclasses
import functools

import jax
import jax.numpy as jnp
from jax import lax
from jax.experimental import pallas as pl
from jax.experimental.pallas import tpu as pltpu
from jax.experimental.pallas import tpu_sc as plsc

N = 50000
E = 800000
IN_FEATURES = 78
HIDDEN = 128
HEADS = 4
OUT_DIM = 128
NUM_GRAPHS = 256

NP = 51200
ET = E + N
EP = 851968
NWT = NP // 32
GP = 272


@functools.cache
def _mesh():
    return plsc.VectorSubcoreMesh(core_axis_name="c", subcore_axis_name="s")


@functools.cache
def _sc_params():
    cp = pltpu.CompilerParams()
    if "needs_layout_passes" in pltpu.CompilerParams.__dataclass_fields__:
        cp = dataclasses.replace(cp, needs_layout_passes=False)
    return cp


def _bcast(v, j):
    idx = jnp.full((16,), 0, jnp.int32) + j
    return v.at[idx].get(mode="promise_in_bounds")


BB = 32


SB = 512


def _msg_body(D, H, hwe, adst, srcs, dsts, offs, zf, agg,
              offv, sidxsb, didxsb, lbuf, gA, gB, zfb, adstb,
              gsemA, gsemB, ssem):
    c = lax.axis_index("c")
    s = lax.axis_index("s")
    w = s * 2 + c
    CH = D // 16
    DEXT = D + 128
    iota = lax.iota(jnp.int32, 16)

    pltpu.sync_copy(offs, offv)
    pltpu.sync_copy(zf, zfb)
    pltpu.sync_copy(adst.at[pl.ds(w * (NWT // 8), NWT // 8)], adstb)

    @pl.loop(0, NWT // 8)
    def _(z):
        pltpu.sync_copy(zfb, agg.at[pl.ds(w * NWT + z * 8, 8)])

    widx = jnp.full((16,), 0, jnp.int32) + w
    lo = plsc.load_gather(offv, [widx])[0]
    hi = plsc.load_gather(offv, [widx + 1])[0]
    al = (lo // SB) * SB
    nsb = (jnp.maximum(hi - al, 0) + SB - 1) // SB
    base_row = w * NWT

    def process(cur, csem, nxt, nsem, j2, stm):
        pltpu.make_async_copy(
            hwe.at[sidxsb.at[pl.ds(j2 * BB, BB)]], cur, csem).wait()

        @pl.when(j2 + 1 < SB // BB)
        def _():
            pltpu.async_copy(
                hwe.at[sidxsb.at[pl.ds((j2 + 1) * BB, BB)]], nxt, nsem)

        for q in range(BB // 16):
            dv = didxsb[pl.ds(j2 * BB + q * 16, 16)]
            lv = jnp.clip(dv - base_row, 0, NWT - 1)
            lbuf[pl.ds(q * 16, 16)] = lv + base_row
            rv = lv >> 3
            cv = (lv & 7) * 16
            for t in range(16):
                e = q * 16 + t
                ge = stm + j2 * BB + e
                valid = jnp.logical_and(ge >= lo, ge < hi)
                vf = jnp.where(valid, jnp.float32(1.0), jnp.float32(0.0))
                bdv = plsc.load_gather(adstb, [_bcast(rv, t), _bcast(cv, t) + iota])
                alpha = cur[e, pl.ds(D, 16)] + bdv
                alpha = jnp.where(alpha > 0, alpha, alpha * jnp.float32(0.2))
                exv = jnp.exp(alpha) * vf
                cur[e, pl.ds(D, 16)] = exv
                for h in range(H):
                    wh = _bcast(exv, h)
                    for k in range(CH // H):
                        col = (h * (CH // H) + k) * 16
                        cur[e, pl.ds(col, 16)] = cur[e, pl.ds(col, 16)] * wh

        pltpu.async_copy(cur, agg.at[lbuf], ssem, add=True).wait()

    @pl.loop(0, nsb)
    def _(m):
        stm = al + m * SB
        pltpu.sync_copy(srcs.at[pl.ds(stm, SB)], sidxsb)
        pltpu.sync_copy(dsts.at[pl.ds(stm, SB)], didxsb)
        pltpu.async_copy(hwe.at[sidxsb.at[pl.ds(0, BB)]], gA, gsemA)

        @pl.loop(0, SB // BB // 2)
        def _(jj):
            process(gA, gsemA, gB, gsemB, 2 * jj, stm)
            process(gB, gsemB, gA, gsemA, 2 * jj + 1, stm)


def _msg_pass(D, H, hwe, adst, srcs, dsts, offs):
    adst = adst.reshape(NP // 8, 8 * 16)[:, :128]
    DEXT = D + 128
    zf = jnp.zeros((8, DEXT), jnp.float32)
    kern = pl.kernel(
        functools.partial(_msg_body, D, H),
        out_type=jax.ShapeDtypeStruct((NP, DEXT), jnp.float32),
        mesh=_mesh(),
        compiler_params=_sc_params(),
        scratch_types=[
            pltpu.VMEM((48,), jnp.int32),
            pltpu.VMEM((SB,), jnp.int32),
            pltpu.VMEM((SB,), jnp.int32),
            pltpu.VMEM((BB,), jnp.int32),
            pltpu.VMEM((BB, DEXT), jnp.float32),
            pltpu.VMEM((BB, DEXT), jnp.float32),
            pltpu.VMEM((8, DEXT), jnp.float32),
            pltpu.VMEM((NWT // 8, 128), jnp.float32),
            pltpu.SemaphoreType.DMA,
            pltpu.SemaphoreType.DMA,
            pltpu.SemaphoreType.DMA,
        ],
    )
    return kern(hwe, adst, srcs, dsts, offs, zf)


NW = NP // 32
BC = 64


def _pool_body(agg3, batch, bst, zsum, zmax, zcnt,
               sum_p, max_p, cnt_p,
               rows, bidx, bstb, sumt, maxt, cntt):
    c = lax.axis_index("c")
    s = lax.axis_index("s")
    w = s * 2 + c
    iota = lax.iota(jnp.int32, 16)

    pltpu.sync_copy(zsum, sumt)
    pltpu.sync_copy(zmax, maxt)
    pltpu.sync_copy(zcnt, cntt)
    pltpu.sync_copy(bst, bstb)

    base = w * NW
    csixteenth = jnp.full((16,), 0.0625, jnp.float32)

    @pl.loop(0, NW // BC)
    def _(j):
        st = base + j * BC
        pltpu.sync_copy(agg3.at[pl.ds(st, BC)], rows)
        pltpu.sync_copy(batch.at[pl.ds(st, BC)], bidx)

        for q in range(BC // 16):
            bv = bidx[pl.ds(q * 16, 16)]
            for t in range(16):
                e = q * 16 + t
                d = _bcast(rows[e, pl.ds(128, 16)], 0) + jnp.float32(1e-16)
                g16 = _bcast(bv, t)
                for k in range(8):
                    cs = pl.ds(k * 16, 16)
                    y = rows[e, cs] / d
                    y = (y + bstb[0, cs]) * bstb[1, cs]
                    y = jnp.maximum(y + bstb[2, cs], jnp.float32(0.0))
                    colv = iota + k * 16
                    plsc.addupdate_scatter(sumt, [g16, colv], y)
                    cur = plsc.load_gather(maxt, [g16, colv])
                    plsc.store_scatter(maxt, [g16, colv], jnp.maximum(cur, y))
                plsc.addupdate_scatter(cntt, [g16, iota], csixteenth)

    pltpu.sync_copy(sumt, sum_p.at[w])
    pltpu.sync_copy(maxt, max_p.at[w])
    pltpu.sync_copy(cntt, cnt_p.at[w])


def _pool_pass(agg3, batch, bst, zsum, zmax, zcnt):
    kern = pl.kernel(
        _pool_body,
        out_type=[
            jax.ShapeDtypeStruct((32, GP, 128), jnp.float32),
            jax.ShapeDtypeStruct((32, GP, 128), jnp.float32),
            jax.ShapeDtypeStruct((32, GP, 16), jnp.float32),
        ],
        mesh=_mesh(),
        compiler_params=_sc_params(),
        scratch_types=[
            pltpu.VMEM((BC, 256), jnp.float32),
            pltpu.VMEM((BC,), jnp.int32),
            pltpu.VMEM((3, 128), jnp.float32),
            pltpu.VMEM((GP, 128), jnp.float32),
            pltpu.VMEM((GP, 128), jnp.float32),
            pltpu.VMEM((GP, 16), jnp.float32),
        ],
    )
    return kern(agg3, batch, bst, zsum, zmax, zcnt)


def _k1_body(x_ref, we_ref, be_ref, w1_ref, a1s_ref, a1d_ref,
             hwe_ref, ad_ref):
    h0 = jnp.maximum(
        jnp.dot(x_ref[...], we_ref[...], preferred_element_type=jnp.float32)
        + be_ref[...], 0.0)
    hw = jnp.dot(h0, w1_ref[...], preferred_element_type=jnp.float32)
    asrc = jnp.dot(hw, a1s_ref[...], preferred_element_type=jnp.float32)
    hwe_ref[...] = jnp.concatenate([hw, asrc], axis=1)
    ad_ref[...] = jnp.dot(hw, a1d_ref[...], preferred_element_type=jnp.float32)


def _k23_body(has_res, agg_ref, denb_ref, res_ref, bias_ref, g1_ref,
              g2_ref, w_ref, as_m_ref, ad_m_ref, hp_ref, hwe_ref, ad_ref):
    b = agg_ref.shape[0]
    nh = agg_ref.shape[1] // HIDDEN
    agg3 = agg_ref[...].reshape(b, nh, HIDDEN)
    den3 = denb_ref[...][:, :nh].reshape(b, nh, 1)
    out = (agg3 / (den3 + 1e-16)).reshape(b, nh * HIDDEN)
    hp = jnp.maximum((out + bias_ref[...]) * g1_ref[...] + g2_ref[...], 0.0)
    if has_res:
        hp = hp + res_ref[...]
    hp_ref[...] = hp
    hw = jnp.dot(hp, w_ref[...], preferred_element_type=jnp.float32)
    asrc = jnp.dot(hw, as_m_ref[...], preferred_element_type=jnp.float32)
    hwe_ref[...] = jnp.concatenate([hw, asrc], axis=1)
    ad_ref[...] = jnp.dot(hw, ad_m_ref[...], preferred_element_type=jnp.float32)


def _k4_body(sum_ref, max_ref, cnt_ref, w1_ref, b1_ref, w2_ref, b2_ref, o_ref):
    stot = jnp.sum(sum_ref[...], axis=0)[:NUM_GRAPHS]
    mtot = jnp.max(max_ref[...], axis=0)[:NUM_GRAPHS]
    cnt = jnp.sum(cnt_ref[...], axis=0)[:NUM_GRAPHS]
    cnt = jnp.sum(cnt, axis=1, keepdims=True)
    mean = stot / jnp.maximum(cnt, 1.0)
    g = jnp.concatenate([mean, mtot], axis=1)
    h = jnp.maximum(
        jnp.dot(g, w1_ref[...], preferred_element_type=jnp.float32) + b1_ref[...],
        0.0)
    o_ref[...] = jnp.dot(h, w2_ref[...], preferred_element_type=jnp.float32) + b2_ref[...]


_BR = 256


def _tc_k1(x_pad, we, be, w1, a1s, a1d):
    return pl.pallas_call(
        _k1_body,
        grid=(NP // _BR,),
        in_specs=[
            pl.BlockSpec((_BR, 128), lambda i: (i, 0)),
            pl.BlockSpec((128, 128), lambda i: (0, 0)),
            pl.BlockSpec((1, 128), lambda i: (0, 0)),
            pl.BlockSpec((128, 4 * HIDDEN), lambda i: (0, 0)),
            pl.BlockSpec((4 * HIDDEN, 128), lambda i: (0, 0)),
            pl.BlockSpec((4 * HIDDEN, 16), lambda i: (0, 0)),
        ],
        out_specs=[
            pl.BlockSpec((_BR, 4 * HIDDEN + 128), lambda i: (i, 0)),
            pl.BlockSpec((_BR, 16), lambda i: (i, 0)),
        ],
        out_shape=[
            jax.ShapeDtypeStruct((NP, 4 * HIDDEN + 128), jnp.float32),
            jax.ShapeDtypeStruct((NP, 16), jnp.float32),
        ],
    )(x_pad, we, be, w1, a1s, a1d)


def _tc_k23(has_res, d_in, d_out, agge, res, bias, g1, g2, w, as_m, ad_m):
    di_blk = d_in // 128
    return pl.pallas_call(
        functools.partial(_k23_body, has_res),
        grid=(NP // _BR,),
        in_specs=[
            pl.BlockSpec((_BR, d_in), lambda i: (i, 0)),
            pl.BlockSpec((_BR, 128), lambda i, n=di_blk: (i, n)),
            pl.BlockSpec((_BR, d_in), lambda i: (i, 0)),
            pl.BlockSpec((1, d_in), lambda i: (0, 0)),
            pl.BlockSpec((1, d_in), lambda i: (0, 0)),
            pl.BlockSpec((1, d_in), lambda i: (0, 0)),
            pl.BlockSpec((d_in, d_out), lambda i: (0, 0)),
            pl.BlockSpec((d_out, 128), lambda i: (0, 0)),
            pl.BlockSpec((d_out, 16), lambda i: (0, 0)),
        ],
        out_specs=[
            pl.BlockSpec((_BR, d_in), lambda i: (i, 0)),
            pl.BlockSpec((_BR, d_out + 128), lambda i: (i, 0)),
            pl.BlockSpec((_BR, 16), lambda i: (i, 0)),
        ],
        out_shape=[
            jax.ShapeDtypeStruct((NP, d_in), jnp.float32),
            jax.ShapeDtypeStruct((NP, d_out + 128), jnp.float32),
            jax.ShapeDtypeStruct((NP, 16), jnp.float32),
        ],
    )(agge, agge, res, bias, g1, g2, w, as_m, ad_m)


def _tc_k4(sum_p, max_p, cnt_p, w1, b1, w2, b2):
    return pl.pallas_call(
        _k4_body,
        out_shape=jax.ShapeDtypeStruct((NUM_GRAPHS, OUT_DIM), jnp.float32),
    )(sum_p, max_p, cnt_p, w1, b1, w2, b2)


def _att_mat(att, width=16):
    h, c = att.shape
    eye = jnp.eye(h, width, dtype=jnp.float32)
    return jnp.einsum("hc,hj->hcj", att, eye).reshape(h * c, width)


def kernel(x, edge_index, batch, params):
    f32 = jnp.float32
    i32 = jnp.int32

    loop = jnp.arange(N, dtype=i32)
    pad = jnp.full((EP - ET,), N, dtype=i32)
    src = jnp.concatenate([edge_index[0].astype(i32), loop, pad])
    dst = jnp.concatenate([edge_index[1].astype(i32), loop, pad])
    order = jnp.argsort(dst)
    srcs = src[order]
    dsts = dst[order]
    offs = jnp.searchsorted(
        dsts, jnp.arange(48, dtype=i32) * NWT, side="left").astype(i32)

    x_pad = jnp.zeros((NP, 128), f32).at[:N, :IN_FEATURES].set(x)
    batch_pad = jnp.full((NP,), NUM_GRAPHS, i32).at[:N].set(batch.astype(i32))

    zsum = jnp.zeros((GP, 128), f32)
    zmax = jnp.full((GP, 128), -1e30, f32)
    zcnt = jnp.zeros((GP, 16), f32)

    p = params
    we = jnp.zeros((128, 128), f32).at[:IN_FEATURES].set(p["W_embed"])
    be = p["b_embed"].reshape(1, 128)

    gat = p["gat"]
    bn = p["bn"]
    inv = 1.0 / jnp.sqrt(1.0 + 1e-5)

    hwe1, ad1 = _tc_k1(x_pad, we, be, gat[0]["W"],
                       _att_mat(gat[0]["att_src"], 128),
                       _att_mat(gat[0]["att_dst"]))
    agge1 = _msg_pass(512, 4, hwe1, ad1, srcs, dsts, offs)

    h1p, hwe2, ad2 = _tc_k23(
        False, 512, 512, agge1, hwe1,
        gat[0]["bias"].reshape(1, 512), (bn[0]["gamma"] * inv).reshape(1, 512),
        bn[0]["beta"].reshape(1, 512), gat[1]["W"],
        _att_mat(gat[1]["att_src"], 128), _att_mat(gat[1]["att_dst"]))
    agge2 = _msg_pass(512, 4, hwe2, ad2, srcs, dsts, offs)

    _, hwe3, ad3 = _tc_k23(
        True, 512, 128, agge2, h1p,
        gat[1]["bias"].reshape(1, 512), (bn[1]["gamma"] * inv).reshape(1, 512),
        bn[1]["beta"].reshape(1, 512), gat[2]["W"],
        _att_mat(gat[2]["att_src"], 128), _att_mat(gat[2]["att_dst"]))
    agge3 = _msg_pass(128, 1, hwe3, ad3, srcs, dsts, offs)

    bst = jnp.stack([
        gat[2]["bias"], bn[2]["gamma"] * inv, bn[2]["beta"]]).astype(f32)
    sum_p, max_p, cnt_p = _pool_pass(agge3, batch_pad, bst, zsum, zmax, zcnt)

    fc = p["fc"]
    return _tc_k4(sum_p, max_p, cnt_p, fc["W1"], fc["b1"].reshape(1, OUT_DIM),
                  fc["W2"], fc["b2"].reshape(1, OUT_DIM))
